# baseline (speedup 1.0000x reference)
import math

import jax
import jax.numpy as jnp
from jax import lax
from jax.experimental import pallas as pl
from jax.experimental.pallas import tpu as pltpu
from jax.experimental.pallas import tpu_sc as plsc

N = 4096
DX = 512
DY = 128
TOPK = 10
L = 40
NROWS = 2458
BLK = 256
MROWS = 2560
NTC = 1792
NTCB = NTC // BLK
MSC = MROWS - NTC
NSCB = MSC // BLK
NEG = -1e30
LN2 = math.log(2.0)
NWORK = 32
RPW = MSC // NWORK
NBUCK = 256

_INVD = tuple(1.0 / math.log2(2.0 + k) if k < TOPK else 0.0 for k in range(L))


def _norm_body(x_ref, y_ref, xn_ref, yn_ref):
    x = x_ref[...]
    nx = jnp.sqrt(jnp.sum(x * x, axis=1, keepdims=True))
    xn_ref[...] = x / jnp.where(nx == 0.0, 1.0, nx)
    y = y_ref[...]
    ny = jnp.sqrt(jnp.sum(y * y, axis=1, keepdims=True))
    yn_ref[...] = y / jnp.where(ny == 0.0, 1.0, ny)


def _sims(xn_ref, xnT_ref, yn_ref, ynT_ref, row0):
    sx = 5.0 * (jax.lax.dot_general(
        xn_ref[...], xnT_ref[...], (((1,), (0,)), ((), ())),
        preferred_element_type=jnp.float32) + 1.0)
    sy = 5.0 * (jax.lax.dot_general(
        yn_ref[...], ynT_ref[...], (((1,), (0,)), ((), ())),
        preferred_element_type=jnp.float32) + 1.0)
    col = jax.lax.broadcasted_iota(jnp.int32, (BLK, N), 1)
    rid = row0 + jax.lax.broadcasted_iota(jnp.int32, (BLK, N), 0)
    return sx, sy, col, col == rid


def _sim_body(xn_ref, xnT_ref, yn_ref, ynT_ref, smx_ref, smy_ref):
    row0 = NTC + pl.program_id(0) * BLK
    sx, sy, _, isdiag = _sims(xn_ref, xnT_ref, yn_ref, ynT_ref, row0)
    smx_ref[...] = jnp.where(isdiag, NEG, sx)
    smy_ref[...] = jnp.where(isdiag, NEG, sy)


def _main_body(xn_ref, xnT_ref, yn_ref, ynT_ref, out_ref,
               sx_ref, sxw_ref, syw_ref):
    b = pl.program_id(0)
    sx, sy, col, isdiag = _sims(xn_ref, xnT_ref, yn_ref, ynT_ref, b * BLK)
    sx_ref[...] = sx
    sxw_ref[...] = jnp.where(isdiag, NEG, sx)
    syw_ref[...] = jnp.where(isdiag, NEG, sy)

    l40 = jax.lax.broadcasted_iota(jnp.int32, (BLK, L), 1)
    l10 = jax.lax.broadcasted_iota(jnp.int32, (BLK, TOPK), 1)

    def step(t, carry):
        y_ss, x_corr = carry
        syw = syw_ref[...]
        m = jnp.max(syw, axis=1, keepdims=True)
        idx = jnp.min(jnp.where(syw == m, col, N), axis=1, keepdims=True)
        hot = col == idx
        xg = jnp.sum(jnp.where(hot, sx_ref[...], 0.0), axis=1, keepdims=True)
        syw_ref[...] = jnp.where(hot, NEG, syw)
        y_ss = jnp.where(l40 == t, m, y_ss)
        x_corr = jnp.where(l40 == t, xg, x_corr)
        return y_ss, x_corr

    def stepx(t, x_ss):
        sxw = sxw_ref[...]
        mx = jnp.max(sxw, axis=1, keepdims=True)
        idxx = jnp.min(jnp.where(sxw == mx, col, N), axis=1, keepdims=True)
        sxw_ref[...] = jnp.where(col == idxx, NEG, sxw)
        return jnp.where(l10 == t, mx, x_ss)

    z = jnp.zeros((BLK, L), jnp.float32)
    y_ss, x_corr = jax.lax.fori_loop(0, L, step, (z, z))
    x_ss = jax.lax.fori_loop(0, TOPK, stepx,
                             jnp.zeros((BLK, TOPK), jnp.float32))

    l40f = l40.astype(jnp.float32)
    invd = jnp.where(l40 < TOPK, math.log(2.0) / jnp.log(2.0 + l40f), 0.0)
    invd10 = math.log(2.0) / jnp.log(2.0 + l10.astype(jnp.float32))
    idcg = jnp.sum((jnp.exp2(x_ss) - 1.0) * invd10, axis=1, keepdims=True)
    inv_idcg = 1.0 / idcg
    g = jnp.exp2(x_corr) - 1.0

    acc = jnp.zeros((BLK, L), jnp.float32)
    for j in range(TOPK):
        yj = y_ss[:, j:j + 1]
        xj = x_corr[:, j:j + 1]
        gj = g[:, j:j + 1]
        pd = yj - y_ss
        frac = -1.0 / (1.0 + jnp.exp(pd))
        dd = (gj - g) * (_INVD[j] - invd)
        acc = acc + jnp.where((xj - x_corr) > 0.0,
                              frac * jnp.abs(dd) * pd, 0.0)
    for k in range(TOPK):
        yk = y_ss[:, k:k + 1]
        xk = x_corr[:, k:k + 1]
        gk = g[:, k:k + 1]
        pd = y_ss - yk
        frac = -1.0 / (1.0 + jnp.exp(pd))
        dd = (g - gk) * (invd - _INVD[k])
        cond = ((x_corr - xk) > 0.0) & (l40 >= TOPK)
        acc = acc + jnp.where(cond, frac * jnp.abs(dd) * pd, 0.0)
    rowsum = jnp.sum(acc, axis=1, keepdims=True) * inv_idcg
    loss_blk = jnp.sum(rowsum)

    @pl.when(b == 0)
    def _():
        out_ref[0, 0] = 0.0
    out_ref[0, 0] += loss_blk


def _sc_body(smx, smy, out, yrow, xrow, candv, candi, hist,
             ysv, ysi, xcv, gbuf, xsv, obuf, invdbuf):
    nc = 2
    wid = lax.axis_index("s") * nc + lax.axis_index("c")
    iota16 = lax.iota(jnp.int32, 16)
    ones16 = jnp.ones(16, jnp.int32)
    zeros16 = jnp.zeros(16, jnp.int32)
    negv = jnp.full((16,), NEG, jnp.float32)
    invd_c = jnp.zeros(16, jnp.float32)
    for k in range(TOPK):
        invd_c = jnp.where(iota16 == k, float(_INVD[k]), invd_c)
    lane0 = iota16 == 0

    def bucket(v):
        vf = jnp.minimum(jnp.maximum((10.0 - v) * (NBUCK / 10.0), 0.0),
                         float(NBUCK - 1))
        return vf.astype(jnp.int32)

    def find_bstar(rowref, ksel):
        def z(i, c):
            hist[pl.ds(i * 16, 16)] = zeros16
            return c
        lax.fori_loop(0, NBUCK // 16, z, 0, unroll=4)

        def h(i, c):
            v = rowref[pl.ds(i * 16, 16)]
            plsc.addupdate_scatter(hist, [bucket(v)], ones16)
            return c
        lax.fori_loop(0, N // 16, h, 0, unroll=16)

        def s(i, carry):
            run, bst = carry
            hh = hist[pl.ds(i * 16, 16)]
            cum = plsc.cumsum(hh) + run
            cand = jnp.where(cum >= ksel, i * 16 + iota16, 1 << 20)
            return cum[15], jnp.minimum(bst, jnp.min(cand))
        _, bst = lax.fori_loop(0, NBUCK // 16, s,
                               (jnp.int32(0), jnp.int32(1 << 20)))
        return bst

    def collect(rowref, bst):
        def c(i, ptr):
            v = rowref[pl.ds(i * 16, 16)]
            m = bucket(v) <= bst
            cnt = plsc.all_reduce_population_count(m)[0]

            @pl.when(cnt > 0)
            def _():
                cum = plsc.cumsum(m.astype(jnp.int32))
                tgt = ptr + cum - 1
                plsc.store_scatter(candv, [tgt], v, mask=m)
                plsc.store_scatter(candi, [tgt], i * 16 + iota16, mask=m)
            return ptr + cnt
        ptr = lax.fori_loop(0, N // 16, c, jnp.int32(0), unroll=8)
        plsc.store_scatter(candv, [ptr + iota16], negv)
        return ptr

    def select(nsel, ptr, want_idx, outv_ref, outi_ref):
        nv = (ptr + 15) // 16

        def t_body(t, c):
            def mx(i, carry):
                mv, pv = carry
                cv = candv[pl.ds(i * 16, 16)]
                upd = cv > mv
                return (jnp.where(upd, cv, mv),
                        jnp.where(upd, i * 16 + iota16, pv))
            mv, pv = lax.fori_loop(0, nv, mx, (negv, zeros16))
            msp = jnp.broadcast_to(jnp.max(mv), (16,))
            psp = jnp.broadcast_to(
                jnp.min(jnp.where(mv == msp, pv, 1 << 20)), (16,))
            tsp = jnp.broadcast_to(t, (16,))
            plsc.store_scatter(outv_ref, [tsp], msp, mask=lane0)
            if want_idx:
                ci = plsc.load_gather(candi, [psp])
                plsc.store_scatter(outi_ref, [tsp], ci, mask=lane0)
            plsc.store_scatter(candv, [psp], negv, mask=lane0)
            return c
        lax.fori_loop(0, nsel, t_body, 0)

    for kk in range(3):
        ysi[pl.ds(kk * 16, 16)] = zeros16
        ysv[pl.ds(kk * 16, 16)] = jnp.zeros(16, jnp.float32)
    invdbuf[...] = invd_c

    def row_body(r, acc16w):
        row = wid * RPW + r
        pltpu.sync_copy(smy.at[row], yrow)
        pltpu.sync_copy(smx.at[row], xrow)

        bsty = find_bstar(yrow, L)
        ptry = collect(yrow, bsty)
        select(L, ptry, True, ysv, ysi)

        bstx = find_bstar(xrow, TOPK)
        ptrx = collect(xrow, bstx)
        select(TOPK, ptrx, False, xsv, None)

        def gat(kk, c):
            idx = ysi[pl.ds(kk * 16, 16)]
            xv = plsc.load_gather(xrow, [idx])
            xcv[pl.ds(kk * 16, 16)] = xv
            gbuf[pl.ds(kk * 16, 16)] = jnp.exp(xv * LN2) - 1.0
            return c
        lax.fori_loop(0, 3, gat, 0)

        xs = xsv[...]
        idcg = jnp.sum(jnp.where(iota16 < TOPK,
                                 (jnp.exp(xs * LN2) - 1.0) * invd_c, 0.0))

        def jsweep(j, acc):
            jsp = jnp.broadcast_to(j, (16,))
            yj = plsc.load_gather(ysv, [jsp])
            xj = plsc.load_gather(xcv, [jsp])
            gj = plsc.load_gather(gbuf, [jsp])
            ivj = plsc.load_gather(invdbuf, [jsp])

            def inner(kk, a):
                kidx = kk * 16 + iota16
                yk = ysv[pl.ds(kk * 16, 16)]
                xk = xcv[pl.ds(kk * 16, 16)]
                gk = gbuf[pl.ds(kk * 16, 16)]
                ivk = jnp.where(kk == 0, invd_c, 0.0)
                pd = yj - yk
                frac = -1.0 / (1.0 + jnp.exp(pd))
                dd = (gj - gk) * (ivj - ivk)
                cond = ((xj - xk) > 0.0) & (kidx < L)
                return a + jnp.where(cond, frac * jnp.abs(dd) * pd, 0.0)
            return lax.fori_loop(0, 3, inner, acc, unroll=3)

        def ksweep(k, acc):
            ksp = jnp.broadcast_to(k, (16,))
            yk = plsc.load_gather(ysv, [ksp])
            xk = plsc.load_gather(xcv, [ksp])
            gk = plsc.load_gather(gbuf, [ksp])
            ivk = plsc.load_gather(invdbuf, [ksp])

            def inner(kk, a):
                jidx = kk * 16 + iota16
                yv = ysv[pl.ds(kk * 16, 16)]
                xv = xcv[pl.ds(kk * 16, 16)]
                gv = gbuf[pl.ds(kk * 16, 16)]
                ivv = jnp.where(kk == 0, invd_c, 0.0)
                pd = yv - yk
                frac = -1.0 / (1.0 + jnp.exp(pd))
                dd = (gv - gk) * (ivv - ivk)
                cond = ((xv - xk) > 0.0) & (jidx >= TOPK) & (jidx < L)
                return a + jnp.where(cond, frac * jnp.abs(dd) * pd, 0.0)
            return lax.fori_loop(0, 3, inner, acc, unroll=3)

        acc = jnp.zeros(16, jnp.float32)
        acc = lax.fori_loop(0, TOPK, jsweep, acc)
        acc = lax.fori_loop(0, TOPK, ksweep, acc)
        keep = jnp.broadcast_to(
            ((NTC + row) < NROWS).astype(jnp.float32), (16,))
        idcg_sp = jnp.broadcast_to(idcg, (16,))
        return acc16w + acc * (keep / idcg_sp)

    acc16w = lax.fori_loop(0, RPW, row_body, jnp.zeros(16, jnp.float32))
    obuf[...] = acc16w
    pltpu.sync_copy(obuf, out.at[wid])


def _impl(x, y):
    xn, yn = pl.pallas_call(
        _norm_body,
        out_shape=[jax.ShapeDtypeStruct((N, DX), jnp.float32),
                   jax.ShapeDtypeStruct((N, DY), jnp.float32)],
    )(x, y)
    xnT = xn.T
    ynT = yn.T

    smx, smy = pl.pallas_call(
        _sim_body,
        grid=(NSCB,),
        in_specs=[
            pl.BlockSpec((BLK, DX), lambda b: (b + NTCB, 0)),
            pl.BlockSpec((DX, N), lambda b: (0, 0)),
            pl.BlockSpec((BLK, DY), lambda b: (b + NTCB, 0)),
            pl.BlockSpec((DY, N), lambda b: (0, 0)),
        ],
        out_specs=[pl.BlockSpec((BLK, N), lambda b: (b, 0)),
                   pl.BlockSpec((BLK, N), lambda b: (b, 0))],
        out_shape=[jax.ShapeDtypeStruct((MSC, N), jnp.float32),
                   jax.ShapeDtypeStruct((MSC, N), jnp.float32)],
    )(xn, xnT, yn, ynT)

    sc = pl.kernel(
        _sc_body,
        out_type=jax.ShapeDtypeStruct((NWORK, 16), jnp.float32),
        mesh=plsc.VectorSubcoreMesh(core_axis_name="c", subcore_axis_name="s"),
        compiler_params=pltpu.CompilerParams(needs_layout_passes=False),
        scratch_types=[
            pltpu.VMEM((N,), jnp.float32),
            pltpu.VMEM((N,), jnp.float32),
            pltpu.VMEM((N + 16,), jnp.float32),
            pltpu.VMEM((N + 16,), jnp.int32),
            pltpu.VMEM((NBUCK,), jnp.int32),
            pltpu.VMEM((48,), jnp.float32),
            pltpu.VMEM((48,), jnp.int32),
            pltpu.VMEM((48,), jnp.float32),
            pltpu.VMEM((48,), jnp.float32),
            pltpu.VMEM((16,), jnp.float32),
            pltpu.VMEM((16,), jnp.float32),
            pltpu.VMEM((16,), jnp.float32),
        ],
    )
    partials = sc(smx, smy)

    loss_tc = pl.pallas_call(
        _main_body,
        grid=(NTCB,),
        in_specs=[
            pl.BlockSpec((BLK, DX), lambda b: (b, 0)),
            pl.BlockSpec((DX, N), lambda b: (0, 0)),
            pl.BlockSpec((BLK, DY), lambda b: (b, 0)),
            pl.BlockSpec((DY, N), lambda b: (0, 0)),
        ],
        out_specs=pl.BlockSpec((1, 1), lambda b: (0, 0),
                               memory_space=pltpu.SMEM),
        out_shape=jax.ShapeDtypeStruct((1, 1), jnp.float32),
        scratch_shapes=[pltpu.VMEM((BLK, N), jnp.float32)] * 3,
    )(xn, xnT, yn, ynT)

    return loss_tc[0, 0] + jnp.sum(partials)


def kernel(x, y):
    return _impl(x, y)
